# baseline (speedup 1.0000x reference)
import jax
import jax.numpy as jnp
from jax.experimental import pallas as pl

_B, _C, _H, _W = 4, 16, 224, 224
_NE = 6


def _conv(x, w, b=None, groups=1):
    out = jax.lax.conv_general_dilated(x, w, (1, 1), ((1, 1), (1, 1)),
                                       dimension_numbers=('NCHW', 'OIHW', 'NCHW'),
                                       feature_group_count=groups)
    if b is not None:
        out = out + b[None, :, None, None]
    return out


def _expert(x, p):
    x = jax.nn.leaky_relu(_conv(x, p['w1'], p['b1']), 0.01)
    x = jax.nn.leaky_relu(_conv(x, p['w2'], p['b2']), 0.01)
    return x


def _sobel(x, wx, wy):
    sx = _conv(x, wx, None, groups=_C)
    sy = _conv(x, wy, None, groups=_C)
    return jnp.abs(sx) + jnp.abs(sy)


def _combine_body(g_ref, e0_ref, e1_ref, e2_ref, e3_ref, e4_ref, e5_ref, o_ref):
    o_ref[...] = (g_ref[0, 0, 0] * e0_ref[...] + g_ref[0, 0, 1] * e1_ref[...]
                  + g_ref[0, 0, 2] * e2_ref[...] + g_ref[0, 0, 3] * e3_ref[...]
                  + g_ref[0, 0, 4] * e4_ref[...] + g_ref[0, 0, 5] * e5_ref[...])


def kernel(vis_h, vis_l, ir_h, ir_l, vis, ir, params):
    loss_coef = 0.01
    p = params
    x_local = jnp.concatenate([vis, ir], axis=1)
    b, c2 = x_local.shape[0], x_local.shape[1]
    xr = x_local.reshape(b, c2, _H // 16, 16, _W // 16, 16)
    pooled = xr.mean(axis=(3, 5)) + xr.max(axis=(3, 5))
    s_local = jax.nn.leaky_relu(pooled.reshape(b, -1), 0.01)
    clean_logits = s_local @ p['w_gate']
    noise_stddev = jax.nn.softplus(s_local @ p['w_noise']) + loss_coef
    noise = jax.random.normal(jax.random.key(1), clean_logits.shape,
                              dtype=clean_logits.dtype)
    logits = clean_logits + noise * noise_stddev
    g = jax.nn.softmax(logits, axis=1)

    E = [
        _expert(vis_h, p['exp_vis_h']),
        _expert(vis_l, p['exp_vis_l']),
        _expert(ir_h, p['exp_ir_h']),
        _expert(ir_l, p['exp_ir_l']),
        _sobel(vis, p['sobel_vis']['wx'], p['sobel_vis']['wy']),
        _sobel(ir, p['sobel_ir']['wx'], p['sobel_ir']['wy']),
    ]

    img_spec = pl.BlockSpec((1, _C, _H, _W), lambda i: (i, 0, 0, 0))
    y = pl.pallas_call(
        _combine_body,
        grid=(_B,),
        in_specs=[pl.BlockSpec((1, 1, _NE), lambda i: (i, 0, 0))] + [img_spec] * 6,
        out_specs=img_spec,
        out_shape=jax.ShapeDtypeStruct((_B, _C, _H, _W), jnp.float32),
    )(g.reshape(_B, 1, _NE), *E)

    importance = g.sum(axis=0)
    loss = (jnp.var(importance, ddof=1) / (importance.mean() ** 2 + 1e-10)) * loss_coef
    return y, loss


# speedup vs baseline: 2.5392x; 2.5392x over previous
import jax
import jax.numpy as jnp
from jax.experimental import pallas as pl
from jax.experimental.pallas import tpu as pltpu

_B, _C, _H, _W = 4, 16, 224, 224
_NE = 6
_LC = 0.01
_RC = 32
_NCH = _H // _RC
_PH = _H + 32
_PW = _W + 2


def _leaky(x):
    return jnp.where(x >= 0, x, _LC * x)


def _pool_sm(x):
    x4 = x.reshape(_C, 14, 16, _W)
    s1 = jnp.sum(x4, axis=2)
    m1 = jnp.max(x4, axis=2)
    s2 = jnp.swapaxes(jnp.sum(jnp.swapaxes(s1, 1, 2).reshape(_C, 14, 16, 14),
                              axis=2), 1, 2)
    m2 = jnp.swapaxes(jnp.max(jnp.swapaxes(m1, 1, 2).reshape(_C, 14, 16, 14),
                              axis=2), 1, 2)
    return s2 * (1.0 / 256.0) + m2


def _fill_padded(dst_ref, x):
    dst_ref[:, 0:16, :] = jnp.zeros((_C, 16, _PW), jnp.bfloat16)
    dst_ref[:, _PH - 16:_PH, :] = jnp.zeros((_C, 16, _PW), jnp.bfloat16)
    zc = jnp.zeros((_C, _H, 1), jnp.bfloat16)
    dst_ref[:, 16:16 + _H, :] = jnp.concatenate(
        [zc, x.astype(jnp.bfloat16), zc], axis=2)


def _im2col(src_ref, r0):
    xs = src_ref[:, pl.ds(r0, 64), :]
    cols = [jax.lax.slice(xs, (0, 15 + dy, dx), (_C, 15 + dy + _RC, dx + _W))
            for dy in range(3) for dx in range(3)]
    return jnp.stack(cols, axis=0).reshape(9 * _C, _RC, _W)


def _conv_layer(src_ref, w_ref, bias_ref, dst_ref):
    dst_ref[:, 0:16, :] = jnp.zeros((_C, 16, _PW), jnp.bfloat16)
    dst_ref[:, _PH - 16:_PH, :] = jnp.zeros((_C, 16, _PW), jnp.bfloat16)

    def chunk(k, carry):
        r0 = k * _RC
        out = jax.lax.dot_general(w_ref[...], _im2col(src_ref, r0),
                                  (((1,), (0,)), ((), ())),
                                  preferred_element_type=jnp.float32)
        out = _leaky(out + bias_ref[...])
        zc = jnp.zeros((_C, _RC, 1), jnp.bfloat16)
        dst_ref[:, pl.ds(r0 + 16, _RC), :] = jnp.concatenate(
            [zc, out.astype(jnp.bfloat16), zc], axis=2)
        return carry
    jax.lax.fori_loop(0, _NCH, chunk, 0, unroll=False)


def _conv_out_accum(src_ref, w_ref, bias_ref, gs, first, y_ref):
    def chunk(k, carry):
        r0 = k * _RC
        out = jax.lax.dot_general(w_ref[...], _im2col(src_ref, r0),
                                  (((1,), (0,)), ((), ())),
                                  preferred_element_type=jnp.float32)
        contrib = gs * _leaky(out + bias_ref[...])
        if first:
            y_ref[0, :, pl.ds(r0, _RC), :] = contrib
        else:
            y_ref[0, :, pl.ds(r0, _RC), :] += contrib
        return carry
    jax.lax.fori_loop(0, _NCH, chunk, 0, unroll=False)


def _sobel_accum(src_ref, wsob_ref, gs, y_ref):
    def chunk(k, carry):
        r0 = k * _RC
        out = jax.lax.dot_general(wsob_ref[...], _im2col(src_ref, r0),
                                  (((1,), (0,)), ((), ())),
                                  preferred_element_type=jnp.float32)
        contrib = gs * (jnp.abs(out[:_C]) + jnp.abs(out[_C:]))
        y_ref[0, :, pl.ds(r0, _RC), :] += contrib
        return carry
    jax.lax.fori_loop(0, _NCH, chunk, 0, unroll=False)


def _fused_body(vh_ref, vl_ref, ihh_ref, il_ref, v_ref, i_ref,
                wgv_ref, wgi_ref, wnv_ref, wni_ref, noise_ref,
                w1a_ref, b1a_ref, w2a_ref, b2a_ref,
                w1b_ref, b1b_ref, w2b_ref, b2b_ref,
                w1c_ref, b1c_ref, w2c_ref, b2c_ref,
                w1d_ref, b1d_ref, w2d_ref, b2d_ref,
                wsob_ref,
                y_ref, loss_ref, pa_ref, pb_ref, imp_ref):
    b = pl.program_id(0)
    xv = v_ref[0]
    xi = i_ref[0]

    pv = _leaky(_pool_sm(xv))
    pi = _leaky(_pool_sm(xi))
    clean = (jnp.sum(pv[..., None] * wgv_ref[...], axis=(0, 1, 2))
             + jnp.sum(pi[..., None] * wgi_ref[...], axis=(0, 1, 2)))
    raw_n = (jnp.sum(pv[..., None] * wnv_ref[...], axis=(0, 1, 2))
             + jnp.sum(pi[..., None] * wni_ref[...], axis=(0, 1, 2)))
    std = jnp.maximum(raw_n, 0.0) + jnp.log1p(jnp.exp(-jnp.abs(raw_n))) + _LC
    logits = clean + noise_ref[0, 0] * std
    ex = jnp.exp(logits - jnp.max(logits))
    g = ex / jnp.sum(ex)

    conv_sets = [
        (vh_ref, w1a_ref, b1a_ref, w2a_ref, b2a_ref),
        (vl_ref, w1b_ref, b1b_ref, w2b_ref, b2b_ref),
        (ihh_ref, w1c_ref, b1c_ref, w2c_ref, b2c_ref),
        (il_ref, w1d_ref, b1d_ref, w2d_ref, b2d_ref),
    ]
    for e, (x_ref, w1, b1, w2, b2) in enumerate(conv_sets):
        _fill_padded(pa_ref, x_ref[0])
        _conv_layer(pa_ref, w1, b1, pb_ref)
        _conv_out_accum(pb_ref, w2, b2, g[e], e == 0, y_ref)

    _fill_padded(pa_ref, xv)
    _sobel_accum(pa_ref, wsob_ref, g[4], y_ref)
    _fill_padded(pa_ref, xi)
    _sobel_accum(pa_ref, wsob_ref, g[5], y_ref)

    imp_new = jnp.where(b == 0, g, imp_ref[0] + g)
    imp_ref[0] = imp_new
    mean = jnp.sum(imp_new) * (1.0 / _NE)
    d = imp_new - mean
    var = jnp.sum(d * d) * (1.0 / (_NE - 1))
    loss_ref[...] = jnp.broadcast_to((var / (mean * mean + 1e-10)) * _LC, (1, 1))


def _wprep(w):
    return jnp.transpose(w, (2, 3, 1, 0)).reshape(144, _C).T.astype(jnp.bfloat16)


def kernel(vis_h, vis_l, ir_h, ir_l, vis, ir, params):
    p = params
    wg = p['w_gate'].reshape(2 * _C, 14, 14, _NE)
    wn = p['w_noise'].reshape(2 * _C, 14, 14, _NE)
    noise = jax.random.normal(jax.random.key(1), (_B, _NE), dtype=jnp.float32)

    eye = jnp.eye(_C, dtype=jnp.float32)
    cwx = p['sobel_vis']['wx'][:, 0].reshape(_C, 9)
    cwy = p['sobel_vis']['wy'][:, 0].reshape(_C, 9)
    wsx = (cwx[:, :, None] * eye[:, None, :]).reshape(_C, 144)
    wsy = (cwy[:, :, None] * eye[:, None, :]).reshape(_C, 144)
    wsob = jnp.concatenate([wsx, wsy], axis=0).astype(jnp.bfloat16)

    img = pl.BlockSpec((1, _C, _H, _W), lambda b: (b, 0, 0, 0))
    full = lambda a: pl.BlockSpec(a.shape, lambda b: (0,) * a.ndim)

    vis_h = vis_h.astype(jnp.bfloat16)
    vis_l = vis_l.astype(jnp.bfloat16)
    ir_h = ir_h.astype(jnp.bfloat16)
    ir_l = ir_l.astype(jnp.bfloat16)

    exp_args = []
    exp_specs = []
    for nm in ('exp_vis_h', 'exp_vis_l', 'exp_ir_h', 'exp_ir_l'):
        w1 = _wprep(p[nm]['w1'])
        b1 = p[nm]['b1'].reshape(_C, 1, 1)
        w2 = _wprep(p[nm]['w2'])
        b2 = p[nm]['b2'].reshape(_C, 1, 1)
        exp_args += [w1, b1, w2, b2]
        exp_specs += [full(w1), full(b1), full(w2), full(b2)]

    args = [vis_h, vis_l, ir_h, ir_l, vis, ir,
            wg[:_C], wg[_C:], wn[:_C], wn[_C:], noise.reshape(_B, 1, _NE),
            *exp_args, wsob]
    specs = ([img] * 6
             + [full(wg[:_C]), full(wg[_C:]), full(wn[:_C]), full(wn[_C:]),
                pl.BlockSpec((1, 1, _NE), lambda b: (b, 0, 0))]
             + exp_specs + [full(wsob)])

    y, loss = pl.pallas_call(
        _fused_body,
        grid=(_B,),
        in_specs=specs,
        out_specs=[img, pl.BlockSpec((1, 1), lambda b: (0, 0))],
        out_shape=[jax.ShapeDtypeStruct((_B, _C, _H, _W), jnp.float32),
                   jax.ShapeDtypeStruct((1, 1), jnp.float32)],
        scratch_shapes=[pltpu.VMEM((_C, _PH, _PW), jnp.bfloat16),
                        pltpu.VMEM((_C, _PH, _PW), jnp.bfloat16),
                        pltpu.VMEM((1, _NE), jnp.float32)],
    )(*args)
    return y, loss[0, 0]


# speedup vs baseline: 2.7064x; 1.0659x over previous
import jax
import jax.numpy as jnp
from jax.experimental import pallas as pl
from jax.experimental.pallas import tpu as pltpu

_B, _C, _H, _W = 4, 16, 224, 224
_NE = 6
_LC = 0.01
_RC = 32
_NCH = _H // _RC
_PH = _H + 32
_PW = _W + 2


def _leaky(x):
    return jnp.where(x >= 0, x, _LC * x)


def _pool_sm(x):
    x4 = x.reshape(_C, 14, 16, _W)
    s1 = jnp.sum(x4, axis=2)
    m1 = jnp.max(x4, axis=2)
    s2 = jnp.swapaxes(jnp.sum(jnp.swapaxes(s1, 1, 2).reshape(_C, 14, 16, 14),
                              axis=2), 1, 2)
    m2 = jnp.swapaxes(jnp.max(jnp.swapaxes(m1, 1, 2).reshape(_C, 14, 16, 14),
                              axis=2), 1, 2)
    return s2 * (1.0 / 256.0) + m2


def _fill_padded(dst_ref, x):
    dst_ref[:, 0:16, :] = jnp.zeros((_C, 16, _PW), jnp.bfloat16)
    dst_ref[:, _PH - 16:_PH, :] = jnp.zeros((_C, 16, _PW), jnp.bfloat16)
    zc = jnp.zeros((_C, _H, 1), jnp.bfloat16)
    dst_ref[:, 16:16 + _H, :] = jnp.concatenate(
        [zc, x.astype(jnp.bfloat16), zc], axis=2)


def _dystack(src_ref, r0):
    xs = src_ref[:, pl.ds(r0, 64), :]
    rows = [jax.lax.slice(xs, (0, 15 + dy, 0), (_C, 15 + dy + _RC, _PW))
            for dy in range(3)]
    return jnp.stack(rows, axis=0).reshape(3 * _C, _RC, _PW)


def _conv_dx(w_ref, x3, m):
    out = jax.lax.dot_general(w_ref[...], x3, (((1,), (0,)), ((), ())),
                              preferred_element_type=jnp.float32)
    return (jax.lax.slice(out, (0, 0, 0), (m, _RC, _W))
            + jax.lax.slice(out, (m, 0, 1), (2 * m, _RC, 1 + _W))
            + jax.lax.slice(out, (2 * m, 0, 2), (3 * m, _RC, 2 + _W)))


def _conv_layer(src_ref, w_ref, bias_ref, dst_ref):
    dst_ref[:, 0:16, :] = jnp.zeros((_C, 16, _PW), jnp.bfloat16)
    dst_ref[:, _PH - 16:_PH, :] = jnp.zeros((_C, 16, _PW), jnp.bfloat16)

    def chunk(k, carry):
        r0 = k * _RC
        out = _conv_dx(w_ref, _dystack(src_ref, r0), _C)
        out = _leaky(out + bias_ref[...])
        zc = jnp.zeros((_C, _RC, 1), jnp.bfloat16)
        dst_ref[:, pl.ds(r0 + 16, _RC), :] = jnp.concatenate(
            [zc, out.astype(jnp.bfloat16), zc], axis=2)
        return carry
    jax.lax.fori_loop(0, _NCH, chunk, 0, unroll=False)


def _conv_out_accum(src_ref, w_ref, bias_ref, gs, first, y_ref):
    def chunk(k, carry):
        r0 = k * _RC
        out = _conv_dx(w_ref, _dystack(src_ref, r0), _C)
        contrib = gs * _leaky(out + bias_ref[...])
        if first:
            y_ref[0, :, pl.ds(r0, _RC), :] = contrib
        else:
            y_ref[0, :, pl.ds(r0, _RC), :] += contrib
        return carry
    jax.lax.fori_loop(0, _NCH, chunk, 0, unroll=False)


def _sobel_accum(src_ref, wsob_ref, gs, y_ref):
    def chunk(k, carry):
        r0 = k * _RC
        out = _conv_dx(wsob_ref, _dystack(src_ref, r0), 2 * _C)
        contrib = gs * (jnp.abs(out[:_C]) + jnp.abs(out[_C:]))
        y_ref[0, :, pl.ds(r0, _RC), :] += contrib
        return carry
    jax.lax.fori_loop(0, _NCH, chunk, 0, unroll=False)


def _fused_body(vh_ref, vl_ref, ihh_ref, il_ref, v_ref, i_ref,
                wgv_ref, wgi_ref, wnv_ref, wni_ref, noise_ref,
                w1a_ref, b1a_ref, w2a_ref, b2a_ref,
                w1b_ref, b1b_ref, w2b_ref, b2b_ref,
                w1c_ref, b1c_ref, w2c_ref, b2c_ref,
                w1d_ref, b1d_ref, w2d_ref, b2d_ref,
                wsob_ref,
                y_ref, loss_ref, pa_ref, pb_ref, imp_ref):
    b = pl.program_id(0)
    xv = v_ref[0]
    xi = i_ref[0]

    pv = _leaky(_pool_sm(xv))
    pi = _leaky(_pool_sm(xi))
    clean = (jnp.sum(pv[..., None] * wgv_ref[...], axis=(0, 1, 2))
             + jnp.sum(pi[..., None] * wgi_ref[...], axis=(0, 1, 2)))
    raw_n = (jnp.sum(pv[..., None] * wnv_ref[...], axis=(0, 1, 2))
             + jnp.sum(pi[..., None] * wni_ref[...], axis=(0, 1, 2)))
    std = jnp.maximum(raw_n, 0.0) + jnp.log1p(jnp.exp(-jnp.abs(raw_n))) + _LC
    logits = clean + noise_ref[0, 0] * std
    ex = jnp.exp(logits - jnp.max(logits))
    g = ex / jnp.sum(ex)

    conv_sets = [
        (vh_ref, w1a_ref, b1a_ref, w2a_ref, b2a_ref),
        (vl_ref, w1b_ref, b1b_ref, w2b_ref, b2b_ref),
        (ihh_ref, w1c_ref, b1c_ref, w2c_ref, b2c_ref),
        (il_ref, w1d_ref, b1d_ref, w2d_ref, b2d_ref),
    ]
    for e, (x_ref, w1, b1, w2, b2) in enumerate(conv_sets):
        _fill_padded(pa_ref, x_ref[0])
        _conv_layer(pa_ref, w1, b1, pb_ref)
        _conv_out_accum(pb_ref, w2, b2, g[e], e == 0, y_ref)

    _fill_padded(pa_ref, xv)
    _sobel_accum(pa_ref, wsob_ref, g[4], y_ref)
    _fill_padded(pa_ref, xi)
    _sobel_accum(pa_ref, wsob_ref, g[5], y_ref)

    imp_new = jnp.where(b == 0, g, imp_ref[0] + g)
    imp_ref[0] = imp_new
    mean = jnp.sum(imp_new) * (1.0 / _NE)
    d = imp_new - mean
    var = jnp.sum(d * d) * (1.0 / (_NE - 1))
    loss_ref[...] = jnp.broadcast_to((var / (mean * mean + 1e-10)) * _LC, (1, 1))


def _wprep(w):
    return jnp.transpose(w, (3, 0, 2, 1)).reshape(48, 48).astype(jnp.bfloat16)


def kernel(vis_h, vis_l, ir_h, ir_l, vis, ir, params):
    p = params
    wg = p['w_gate'].reshape(2 * _C, 14, 14, _NE)
    wn = p['w_noise'].reshape(2 * _C, 14, 14, _NE)
    noise = jax.random.normal(jax.random.key(1), (_B, _NE), dtype=jnp.float32)

    eye = jnp.eye(_C, dtype=jnp.float32)
    cwx = p['sobel_vis']['wx'][:, 0]
    cwy = p['sobel_vis']['wy'][:, 0]
    cw = jnp.stack([cwx, cwy], axis=0)
    t = (jnp.transpose(cw, (3, 0, 1, 2))[:, :, :, :, None]
         * eye[None, None, :, None, :])
    wsob = t.reshape(96, 48).astype(jnp.bfloat16)

    img = pl.BlockSpec((1, _C, _H, _W), lambda b: (b, 0, 0, 0))
    full = lambda a: pl.BlockSpec(a.shape, lambda b: (0,) * a.ndim)

    vis_h = vis_h.astype(jnp.bfloat16)
    vis_l = vis_l.astype(jnp.bfloat16)
    ir_h = ir_h.astype(jnp.bfloat16)
    ir_l = ir_l.astype(jnp.bfloat16)

    exp_args = []
    exp_specs = []
    for nm in ('exp_vis_h', 'exp_vis_l', 'exp_ir_h', 'exp_ir_l'):
        w1 = _wprep(p[nm]['w1'])
        b1 = p[nm]['b1'].reshape(_C, 1, 1)
        w2 = _wprep(p[nm]['w2'])
        b2 = p[nm]['b2'].reshape(_C, 1, 1)
        exp_args += [w1, b1, w2, b2]
        exp_specs += [full(w1), full(b1), full(w2), full(b2)]

    args = [vis_h, vis_l, ir_h, ir_l, vis, ir,
            wg[:_C], wg[_C:], wn[:_C], wn[_C:], noise.reshape(_B, 1, _NE),
            *exp_args, wsob]
    specs = ([img] * 6
             + [full(wg[:_C]), full(wg[_C:]), full(wn[:_C]), full(wn[_C:]),
                pl.BlockSpec((1, 1, _NE), lambda b: (b, 0, 0))]
             + exp_specs + [full(wsob)])

    y, loss = pl.pallas_call(
        _fused_body,
        grid=(_B,),
        in_specs=specs,
        out_specs=[img, pl.BlockSpec((1, 1), lambda b: (0, 0))],
        out_shape=[jax.ShapeDtypeStruct((_B, _C, _H, _W), jnp.float32),
                   jax.ShapeDtypeStruct((1, 1), jnp.float32)],
        scratch_shapes=[pltpu.VMEM((_C, _PH, _PW), jnp.bfloat16),
                        pltpu.VMEM((_C, _PH, _PW), jnp.bfloat16),
                        pltpu.VMEM((1, _NE), jnp.float32)],
    )(*args)
    return y, loss[0, 0]
